# Initial kernel scaffold; baseline (speedup 1.0000x reference)
import jax
import jax.numpy as jnp
from jax.experimental import pallas as pl


def kernel(node_features, edge_index, W, b, hop_weights):
    raise NotImplementedError("write your pallas kernel here")


# speedup vs baseline: 22.0663x; 22.0663x over previous
import jax
import jax.numpy as jnp
from jax import lax
from jax.experimental import pallas as pl
from jax.experimental.pallas import tpu as pltpu
from jax.experimental.pallas import tpu_sc as plsc

N = 10000
E = 320000
D = 128
H = 8
NC = 2
NS = 16
NW = NC * NS
EPW = E // NW
CH = 128
NCHUNK = EPW // CH
REM = EPW - NCHUNK * CH
RPS = 632
RPSB = 640
NPAD = NS * RPS
DW = 16
BLK = 1000
NBLK = N // BLK

import functools


@functools.cache
def _mesh():
    return plsc.VectorSubcoreMesh(
        core_axis_name="c", subcore_axis_name="s", num_cores=NC, num_subcores=NS
    )


def _zero_fill(ref, nrows, ncols):
    cpr = ncols // 16

    def body(i, carry):
        r = i // cpr
        c = (i % cpr) * 16
        ref[r, pl.ds(c, 16)] = jnp.zeros((16,), jnp.float32)
        return carry

    lax.fori_loop(0, nrows * cpr, body, 0)


def _one_fill(ref, nrows):
    def body(i, carry):
        ref[i, pl.ds(0, 16)] = jnp.ones((16,), jnp.float32)
        return carry

    lax.fori_loop(0, nrows, body, 0)


def _worker(cid, sid):
    base = (sid * NC + cid) * EPW
    r0 = sid * RPS
    return base, r0


def _fill_1d(ref, n, value):
    def body(i, carry):
        ref[pl.ds(i * 16, 16)] = jnp.full((16,), value, jnp.float32)
        return carry

    lax.fori_loop(0, n // 16, body, 0)


def _sc_degree_body(dst_hbm, pdeg_hbm, idx_d, idx_d2, ones, ones2, dbuf, dacc):
    cid = lax.axis_index("c")
    sid = lax.axis_index("s")
    base, r0 = _worker(cid, sid)

    _fill_1d(dbuf, RPSB, 0.0)
    _fill_1d(ones, CH, 1.0)
    _fill_1d(ones2, REM, 1.0)
    pltpu.sync_copy(dbuf.at[pl.ds(0, RPS)], dacc.at[pl.ds(r0, RPS)])

    plsc.subcore_barrier()

    @pl.loop(0, NCHUNK)
    def chunk(j):
        off = base + j * CH
        pltpu.sync_copy(dst_hbm.at[pl.ds(off, CH)], idx_d)
        pltpu.sync_copy(ones, dacc.at[idx_d], add=True)

    offr = base + NCHUNK * CH
    pltpu.sync_copy(dst_hbm.at[pl.ds(offr, REM)], idx_d2)
    pltpu.sync_copy(ones2, dacc.at[idx_d2], add=True)

    plsc.subcore_barrier()

    pltpu.sync_copy(dacc.at[pl.ds(r0, RPS)], dbuf.at[pl.ds(0, RPS)])
    pltpu.sync_copy(dbuf.at[pl.ds(0, RPS)],
                    pdeg_hbm.at[pl.ds(cid * NPAD + r0, RPS)])


@functools.cache
def _sc_degree():
    return pl.kernel(
        _sc_degree_body,
        out_type=[jax.ShapeDtypeStruct((NC * NPAD,), jnp.float32)],
        mesh=_mesh(),
        scratch_types=[
            pltpu.VMEM((CH,), jnp.int32),
            pltpu.VMEM((REM,), jnp.int32),
            pltpu.VMEM((CH,), jnp.float32),
            pltpu.VMEM((REM,), jnp.float32),
            pltpu.VMEM((RPSB,), jnp.float32),
            pltpu.VMEM_SHARED((NPAD,), jnp.float32),
        ],
    )


def _sc_agg_body(t_hbm, src_hbm, dst_hbm, psum_hbm,
                 idx_s, idx_d, rows, idx_s2, idx_d2, rows2, zbuf, acc, sem):
    cid = lax.axis_index("c")
    sid = lax.axis_index("s")
    base, r0 = _worker(cid, sid)

    _zero_fill(zbuf, 8, D)

    @pl.loop(0, RPS // 8)
    def zrow(k):
        pltpu.sync_copy(zbuf, acc.at[pl.ds(r0 + k * 8, 8)])

    plsc.subcore_barrier()

    @pl.loop(0, NCHUNK)
    def chunk(j):
        off = base + j * CH
        pltpu.sync_copy(src_hbm.at[pl.ds(off, CH)], idx_s)
        pltpu.sync_copy(dst_hbm.at[pl.ds(off, CH)], idx_d)
        pltpu.async_copy(t_hbm.at[idx_s], rows, sem).wait()
        pltpu.sync_copy(rows, acc.at[idx_d], add=True)

    offr = base + NCHUNK * CH
    pltpu.sync_copy(src_hbm.at[pl.ds(offr, REM)], idx_s2)
    pltpu.sync_copy(dst_hbm.at[pl.ds(offr, REM)], idx_d2)
    pltpu.async_copy(t_hbm.at[idx_s2], rows2, sem).wait()
    pltpu.sync_copy(rows2, acc.at[idx_d2], add=True)

    plsc.subcore_barrier()

    @pl.loop(0, RPS // 8)
    def orow(k):
        pltpu.sync_copy(acc.at[pl.ds(r0 + k * 8, 8)], zbuf)
        pltpu.sync_copy(zbuf, psum_hbm.at[pl.ds(cid * NPAD + r0 + k * 8, 8)])


@functools.cache
def _sc_agg():
    return pl.kernel(
        _sc_agg_body,
        out_type=[jax.ShapeDtypeStruct((NC * NPAD, D), jnp.float32)],
        mesh=_mesh(),
        scratch_types=[
            pltpu.VMEM((CH,), jnp.int32),
            pltpu.VMEM((CH,), jnp.int32),
            pltpu.VMEM((CH, D), jnp.float32),
            pltpu.VMEM((REM,), jnp.int32),
            pltpu.VMEM((REM,), jnp.int32),
            pltpu.VMEM((REM, D), jnp.float32),
            pltpu.VMEM((8, D), jnp.float32),
            pltpu.VMEM_SHARED((NPAD, D), jnp.float32),
            pltpu.SemaphoreType.DMA,
        ],
    )


def _mean_wb(W_ref, b_ref):
    Wbar = W_ref[0]
    bbar = b_ref[0]
    for h in range(1, H):
        Wbar = Wbar + W_ref[h]
        bbar = bbar + b_ref[h]
    return Wbar * (1.0 / H), bbar * (1.0 / H)


def _matxw(x, Wbar, bbar):
    y = lax.dot_general(x, Wbar, (((1,), (1,)), ((), ())),
                        preferred_element_type=jnp.float32)
    return y + bbar[None, :]


def _tc_transform(x, W, b):
    def body(W_ref, b_ref, x_ref, o_ref):
        Wbar, bbar = _mean_wb(W_ref, b_ref)
        o_ref[...] = _matxw(x_ref[...], Wbar, bbar)

    return pl.pallas_call(
        body,
        grid=(NBLK,),
        in_specs=[
            pl.BlockSpec((H, D, D), lambda i: (0, 0, 0)),
            pl.BlockSpec((H, D), lambda i: (0, 0)),
            pl.BlockSpec((BLK, D), lambda i: (i, 0)),
        ],
        out_specs=pl.BlockSpec((BLK, D), lambda i: (i, 0)),
        out_shape=jax.ShapeDtypeStruct((N, D), jnp.float32),
    )(W, b, x)


def _deg_inv(dp_ref):
    i = pl.program_id(0)
    deg = dp_ref[0, i] + dp_ref[1, i]
    return 1.0 / jnp.maximum(deg, 1.0)


def _tc_combine(p, dp, W, b):
    def body(W_ref, b_ref, p_ref, dp_ref, hop_ref, t_ref):
        Wbar, bbar = _mean_wb(W_ref, b_ref)
        invd = _deg_inv(dp_ref)
        hop = (p_ref[0] + p_ref[1]) * invd[:, None]
        hop_ref[...] = hop
        t_ref[...] = _matxw(hop, Wbar, bbar)

    return pl.pallas_call(
        body,
        grid=(NBLK,),
        in_specs=[
            pl.BlockSpec((H, D, D), lambda i: (0, 0, 0)),
            pl.BlockSpec((H, D), lambda i: (0, 0)),
            pl.BlockSpec((NC, BLK, D), lambda i: (0, i, 0)),
            pl.BlockSpec((NC, NBLK, BLK), lambda i: (0, 0, 0)),
        ],
        out_specs=[
            pl.BlockSpec((BLK, D), lambda i: (i, 0)),
            pl.BlockSpec((BLK, D), lambda i: (i, 0)),
        ],
        out_shape=[
            jax.ShapeDtypeStruct((N, D), jnp.float32),
            jax.ShapeDtypeStruct((N, D), jnp.float32),
        ],
    )(W, b, p, dp)


def _tc_fuse(hw, hop1, p2, dp):
    def body(hw_ref, hop1_ref, p_ref, dp_ref, o_ref):
        invd = _deg_inv(dp_ref)
        hop2 = (p_ref[0] + p_ref[1]) * invd[:, None]
        a0 = hw_ref[0]
        a1 = hw_ref[1]
        m = jnp.maximum(a0, a1)
        e0 = jnp.exp(a0 - m)
        e1 = jnp.exp(a1 - m)
        inv = 1.0 / (e0 + e1)
        o_ref[...] = (e0 * inv) * hop1_ref[...] + (e1 * inv) * hop2

    return pl.pallas_call(
        body,
        grid=(NBLK,),
        in_specs=[
            pl.BlockSpec(memory_space=pltpu.SMEM),
            pl.BlockSpec((BLK, D), lambda i: (i, 0)),
            pl.BlockSpec((NC, BLK, D), lambda i: (0, i, 0)),
            pl.BlockSpec((NC, NBLK, BLK), lambda i: (0, 0, 0)),
        ],
        out_specs=pl.BlockSpec((BLK, D), lambda i: (i, 0)),
        out_shape=jax.ShapeDtypeStruct((N, D), jnp.float32),
    )(hw, hop1, p2, dp)


@jax.jit
def kernel(node_features, edge_index, W, b, hop_weights):
    src = edge_index[0]
    dst = edge_index[1]
    dp = _sc_degree()(dst)[0].reshape(NC, NPAD)[:, :N].reshape(NC, NBLK, BLK)
    t1 = _tc_transform(node_features, W, b)
    p1 = _sc_agg()(t1, src, dst)[0].reshape(NC, NPAD, D)
    hop1, t2 = _tc_combine(p1, dp, W, b)
    p2 = _sc_agg()(t2, src, dst)[0].reshape(NC, NPAD, D)
    return _tc_fuse(hop_weights, hop1, p2, dp)


# speedup vs baseline: 31.8493x; 1.4433x over previous
import jax
import jax.numpy as jnp
from jax import lax
from jax.experimental import pallas as pl
from jax.experimental.pallas import tpu as pltpu
from jax.experimental.pallas import tpu_sc as plsc

N = 10000
E = 320000
D = 128
H = 8
NC = 2
NS = 16
NW = NC * NS
EPW = E // NW
CH = 128
NCHUNK = EPW // CH
REM = EPW - NCHUNK * CH
RPS = 632
RPSB = 640
NPAD = NS * RPS
DW = 16
BLK = 1000
NBLK = N // BLK

import functools


@functools.cache
def _mesh():
    return plsc.VectorSubcoreMesh(
        core_axis_name="c", subcore_axis_name="s", num_cores=NC, num_subcores=NS
    )


def _zero_fill(ref, nrows, ncols):
    cpr = ncols // 16

    def body(i, carry):
        r = i // cpr
        c = (i % cpr) * 16
        ref[r, pl.ds(c, 16)] = jnp.zeros((16,), jnp.float32)
        return carry

    lax.fori_loop(0, nrows * cpr, body, 0)


def _one_fill(ref, nrows):
    def body(i, carry):
        ref[i, pl.ds(0, 16)] = jnp.ones((16,), jnp.float32)
        return carry

    lax.fori_loop(0, nrows, body, 0)


def _worker(cid, sid):
    base = (sid * NC + cid) * EPW
    r0 = sid * RPS
    return base, r0


def _fill_1d(ref, n, value):
    def body(i, carry):
        ref[pl.ds(i * 16, 16)] = jnp.full((16,), value, jnp.float32)
        return carry

    lax.fori_loop(0, n // 16, body, 0)


def _sc_degree_body(dst_hbm, pdeg_hbm, idx_d, idx_d2, ones, ones2, dbuf, dacc):
    cid = lax.axis_index("c")
    sid = lax.axis_index("s")
    base, r0 = _worker(cid, sid)

    _fill_1d(dbuf, RPSB, 0.0)
    _fill_1d(ones, CH, 1.0)
    _fill_1d(ones2, REM, 1.0)
    pltpu.sync_copy(dbuf.at[pl.ds(0, RPS)], dacc.at[pl.ds(r0, RPS)])

    plsc.subcore_barrier()

    @pl.loop(0, NCHUNK)
    def chunk(j):
        off = base + j * CH
        pltpu.sync_copy(dst_hbm.at[pl.ds(off, CH)], idx_d)
        pltpu.sync_copy(ones, dacc.at[idx_d], add=True)

    offr = base + NCHUNK * CH
    pltpu.sync_copy(dst_hbm.at[pl.ds(offr, REM)], idx_d2)
    pltpu.sync_copy(ones2, dacc.at[idx_d2], add=True)

    plsc.subcore_barrier()

    pltpu.sync_copy(dacc.at[pl.ds(r0, RPS)], dbuf.at[pl.ds(0, RPS)])
    pltpu.sync_copy(dbuf.at[pl.ds(0, RPS)],
                    pdeg_hbm.at[pl.ds(cid * NPAD + r0, RPS)])


@functools.cache
def _sc_degree():
    return pl.kernel(
        _sc_degree_body,
        out_type=[jax.ShapeDtypeStruct((NC * NPAD,), jnp.float32)],
        mesh=_mesh(),
        scratch_types=[
            pltpu.VMEM((CH,), jnp.int32),
            pltpu.VMEM((REM,), jnp.int32),
            pltpu.VMEM((CH,), jnp.float32),
            pltpu.VMEM((REM,), jnp.float32),
            pltpu.VMEM((RPSB,), jnp.float32),
            pltpu.VMEM_SHARED((NPAD,), jnp.float32),
        ],
    )


def _sc_agg_body(t_hbm, src_hbm, dst_hbm, psum_hbm,
                 idx_s0, idx_d0, idx_s1, idx_d1, rows0, rows1,
                 idx_s2, idx_d2, rows2, zbuf, acc, sem0, sem1):
    cid = lax.axis_index("c")
    sid = lax.axis_index("s")
    base, r0 = _worker(cid, sid)
    idx_s = (idx_s0, idx_s1)
    idx_d = (idx_d0, idx_d1)
    rows = (rows0, rows1)
    sems = (sem0, sem1)

    _zero_fill(zbuf, 8, D)

    @pl.loop(0, RPS // 8)
    def zrow(k):
        pltpu.sync_copy(zbuf, acc.at[pl.ds(r0 + k * 8, 8)])

    plsc.subcore_barrier()

    pltpu.sync_copy(src_hbm.at[pl.ds(base, CH)], idx_s0)
    pltpu.sync_copy(dst_hbm.at[pl.ds(base, CH)], idx_d0)
    pltpu.async_copy(t_hbm.at[idx_s0], rows0, sem0)

    @pl.loop(0, NCHUNK, step=2)
    def pair(i):
        for bb in range(2):
            j = i + bb
            nb = 1 - bb
            jn = j + 1

            @pl.when(jn < NCHUNK)
            def _():
                off = base + jn * CH
                pltpu.sync_copy(src_hbm.at[pl.ds(off, CH)], idx_s[nb])
                pltpu.sync_copy(dst_hbm.at[pl.ds(off, CH)], idx_d[nb])
                pltpu.async_copy(t_hbm.at[idx_s[nb]], rows[nb], sems[nb])

            pltpu.make_async_copy(t_hbm.at[idx_s[bb]], rows[bb], sems[bb]).wait()
            pltpu.sync_copy(rows[bb], acc.at[idx_d[bb]], add=True)

    offr = base + NCHUNK * CH
    pltpu.sync_copy(src_hbm.at[pl.ds(offr, REM)], idx_s2)
    pltpu.sync_copy(dst_hbm.at[pl.ds(offr, REM)], idx_d2)
    pltpu.async_copy(t_hbm.at[idx_s2], rows2, sem0).wait()
    pltpu.sync_copy(rows2, acc.at[idx_d2], add=True)

    plsc.subcore_barrier()

    @pl.loop(0, RPS // 8)
    def orow(k):
        pltpu.sync_copy(acc.at[pl.ds(r0 + k * 8, 8)], zbuf)
        pltpu.sync_copy(zbuf, psum_hbm.at[pl.ds(cid * NPAD + r0 + k * 8, 8)])


@functools.cache
def _sc_agg():
    return pl.kernel(
        _sc_agg_body,
        out_type=[jax.ShapeDtypeStruct((NC * NPAD, D), jnp.float32)],
        mesh=_mesh(),
        scratch_types=[
            pltpu.VMEM((CH,), jnp.int32),
            pltpu.VMEM((CH,), jnp.int32),
            pltpu.VMEM((CH,), jnp.int32),
            pltpu.VMEM((CH,), jnp.int32),
            pltpu.VMEM((CH, D), jnp.float32),
            pltpu.VMEM((CH, D), jnp.float32),
            pltpu.VMEM((REM,), jnp.int32),
            pltpu.VMEM((REM,), jnp.int32),
            pltpu.VMEM((REM, D), jnp.float32),
            pltpu.VMEM((8, D), jnp.float32),
            pltpu.VMEM_SHARED((NPAD, D), jnp.float32),
            pltpu.SemaphoreType.DMA,
            pltpu.SemaphoreType.DMA,
        ],
    )


def _mean_wb(W_ref, b_ref):
    Wbar = W_ref[0]
    bbar = b_ref[0]
    for h in range(1, H):
        Wbar = Wbar + W_ref[h]
        bbar = bbar + b_ref[h]
    return Wbar * (1.0 / H), bbar * (1.0 / H)


def _matxw(x, Wbar, bbar):
    y = lax.dot_general(x, Wbar, (((1,), (1,)), ((), ())),
                        preferred_element_type=jnp.float32)
    return y + bbar[None, :]


def _tc_transform(x, W, b):
    def body(W_ref, b_ref, x_ref, o_ref):
        Wbar, bbar = _mean_wb(W_ref, b_ref)
        o_ref[...] = _matxw(x_ref[...], Wbar, bbar)

    return pl.pallas_call(
        body,
        grid=(NBLK,),
        in_specs=[
            pl.BlockSpec((H, D, D), lambda i: (0, 0, 0)),
            pl.BlockSpec((H, D), lambda i: (0, 0)),
            pl.BlockSpec((BLK, D), lambda i: (i, 0)),
        ],
        out_specs=pl.BlockSpec((BLK, D), lambda i: (i, 0)),
        out_shape=jax.ShapeDtypeStruct((N, D), jnp.float32),
    )(W, b, x)


def _deg_inv(dp_ref):
    i = pl.program_id(0)
    deg = dp_ref[0, i] + dp_ref[1, i]
    return 1.0 / jnp.maximum(deg, 1.0)


def _tc_combine(p, dp, W, b):
    def body(W_ref, b_ref, p_ref, dp_ref, hop_ref, t_ref):
        Wbar, bbar = _mean_wb(W_ref, b_ref)
        invd = _deg_inv(dp_ref)
        hop = (p_ref[0] + p_ref[1]) * invd[:, None]
        hop_ref[...] = hop
        t_ref[...] = _matxw(hop, Wbar, bbar)

    return pl.pallas_call(
        body,
        grid=(NBLK,),
        in_specs=[
            pl.BlockSpec((H, D, D), lambda i: (0, 0, 0)),
            pl.BlockSpec((H, D), lambda i: (0, 0)),
            pl.BlockSpec((NC, BLK, D), lambda i: (0, i, 0)),
            pl.BlockSpec((NC, NBLK, BLK), lambda i: (0, 0, 0)),
        ],
        out_specs=[
            pl.BlockSpec((BLK, D), lambda i: (i, 0)),
            pl.BlockSpec((BLK, D), lambda i: (i, 0)),
        ],
        out_shape=[
            jax.ShapeDtypeStruct((N, D), jnp.float32),
            jax.ShapeDtypeStruct((N, D), jnp.float32),
        ],
    )(W, b, p, dp)


def _tc_fuse(hw, hop1, p2, dp):
    def body(hw_ref, hop1_ref, p_ref, dp_ref, o_ref):
        invd = _deg_inv(dp_ref)
        hop2 = (p_ref[0] + p_ref[1]) * invd[:, None]
        a0 = hw_ref[0]
        a1 = hw_ref[1]
        m = jnp.maximum(a0, a1)
        e0 = jnp.exp(a0 - m)
        e1 = jnp.exp(a1 - m)
        inv = 1.0 / (e0 + e1)
        o_ref[...] = (e0 * inv) * hop1_ref[...] + (e1 * inv) * hop2

    return pl.pallas_call(
        body,
        grid=(NBLK,),
        in_specs=[
            pl.BlockSpec(memory_space=pltpu.SMEM),
            pl.BlockSpec((BLK, D), lambda i: (i, 0)),
            pl.BlockSpec((NC, BLK, D), lambda i: (0, i, 0)),
            pl.BlockSpec((NC, NBLK, BLK), lambda i: (0, 0, 0)),
        ],
        out_specs=pl.BlockSpec((BLK, D), lambda i: (i, 0)),
        out_shape=jax.ShapeDtypeStruct((N, D), jnp.float32),
    )(hw, hop1, p2, dp)


@jax.jit
def kernel(node_features, edge_index, W, b, hop_weights):
    src = edge_index[0]
    dst = edge_index[1]
    dp = _sc_degree()(dst)[0].reshape(NC, NPAD)[:, :N].reshape(NC, NBLK, BLK)
    t1 = _tc_transform(node_features, W, b)
    p1 = _sc_agg()(t1, src, dst)[0].reshape(NC, NPAD, D)
    hop1, t2 = _tc_combine(p1, dp, W, b)
    p2 = _sc_agg()(t2, src, dst)[0].reshape(NC, NPAD, D)
    return _tc_fuse(hop_weights, hop1, p2, dp)


# speedup vs baseline: 33.9237x; 1.0651x over previous
import jax
import jax.numpy as jnp
from jax import lax
from jax.experimental import pallas as pl
from jax.experimental.pallas import tpu as pltpu
from jax.experimental.pallas import tpu_sc as plsc

N = 10000
E = 320000
D = 128
H = 8
NC = 2
NS = 16
NW = NC * NS
EPW = E // NW
CH = 128
NCHUNK = EPW // CH
REM = EPW - NCHUNK * CH
ZR = 64
ZF = 632 // ZR
ZT = 632 - ZF * ZR
RPS = 632
RPSB = 640
NPAD = NS * RPS
DW = 16
BLK = 1000
NBLK = N // BLK

import functools


@functools.cache
def _mesh():
    return plsc.VectorSubcoreMesh(
        core_axis_name="c", subcore_axis_name="s", num_cores=NC, num_subcores=NS
    )


def _zero_fill(ref, nrows, ncols):
    cpr = ncols // 16

    def body(i, carry):
        r = i // cpr
        c = (i % cpr) * 16
        ref[r, pl.ds(c, 16)] = jnp.zeros((16,), jnp.float32)
        return carry

    lax.fori_loop(0, nrows * cpr, body, 0)


def _one_fill(ref, nrows):
    def body(i, carry):
        ref[i, pl.ds(0, 16)] = jnp.ones((16,), jnp.float32)
        return carry

    lax.fori_loop(0, nrows, body, 0)


def _worker(cid, sid):
    base = (sid * NC + cid) * EPW
    r0 = sid * RPS
    return base, r0


def _fill_1d(ref, n, value):
    def body(i, carry):
        ref[pl.ds(i * 16, 16)] = jnp.full((16,), value, jnp.float32)
        return carry

    lax.fori_loop(0, n // 16, body, 0)


def _sc_degree_body(dst_hbm, pdeg_hbm, idx_d, idx_d2, ones, ones2, dbuf, dacc):
    cid = lax.axis_index("c")
    sid = lax.axis_index("s")
    base, r0 = _worker(cid, sid)

    _fill_1d(dbuf, RPSB, 0.0)
    _fill_1d(ones, CH, 1.0)
    _fill_1d(ones2, REM, 1.0)
    pltpu.sync_copy(dbuf.at[pl.ds(0, RPS)], dacc.at[pl.ds(r0, RPS)])

    plsc.subcore_barrier()

    @pl.loop(0, NCHUNK)
    def chunk(j):
        off = base + j * CH
        pltpu.sync_copy(dst_hbm.at[pl.ds(off, CH)], idx_d)
        pltpu.sync_copy(ones, dacc.at[idx_d], add=True)

    offr = base + NCHUNK * CH
    pltpu.sync_copy(dst_hbm.at[pl.ds(offr, REM)], idx_d2)
    pltpu.sync_copy(ones2, dacc.at[idx_d2], add=True)

    plsc.subcore_barrier()

    pltpu.sync_copy(dacc.at[pl.ds(r0, RPS)], dbuf.at[pl.ds(0, RPS)])
    pltpu.sync_copy(dbuf.at[pl.ds(0, RPS)],
                    pdeg_hbm.at[pl.ds(cid * NPAD + r0, RPS)])


@functools.cache
def _sc_degree():
    return pl.kernel(
        _sc_degree_body,
        out_type=[jax.ShapeDtypeStruct((NC * NPAD,), jnp.float32)],
        mesh=_mesh(),
        scratch_types=[
            pltpu.VMEM((CH,), jnp.int32),
            pltpu.VMEM((REM,), jnp.int32),
            pltpu.VMEM((CH,), jnp.float32),
            pltpu.VMEM((REM,), jnp.float32),
            pltpu.VMEM((RPSB,), jnp.float32),
            pltpu.VMEM_SHARED((NPAD,), jnp.float32),
        ],
    )


def _sc_agg_body(t_hbm, src_hbm, dst_hbm, psum_hbm,
                 idx_s0, idx_d0, idx_s1, idx_d1, rows0, rows1,
                 idx_s2, idx_d2, rows2, zbuf, acc, sem0, sem1):
    cid = lax.axis_index("c")
    sid = lax.axis_index("s")
    base, r0 = _worker(cid, sid)
    idx_s = (idx_s0, idx_s1)
    idx_d = (idx_d0, idx_d1)
    rows = (rows0, rows1)
    sems = (sem0, sem1)

    _zero_fill(zbuf, ZR, D)

    @pl.loop(0, ZF)
    def zrow(k):
        pltpu.sync_copy(zbuf, acc.at[pl.ds(r0 + k * ZR, ZR)])

    pltpu.sync_copy(zbuf.at[pl.ds(0, ZT)], acc.at[pl.ds(r0 + ZF * ZR, ZT)])

    plsc.subcore_barrier()

    pltpu.sync_copy(src_hbm.at[pl.ds(base, CH)], idx_s0)
    pltpu.sync_copy(dst_hbm.at[pl.ds(base, CH)], idx_d0)
    pltpu.async_copy(t_hbm.at[idx_s0], rows0, sem0)

    @pl.loop(0, NCHUNK, step=2)
    def pair(i):
        for bb in range(2):
            j = i + bb
            nb = 1 - bb
            jn = j + 1

            @pl.when(jn < NCHUNK)
            def _():
                off = base + jn * CH
                pltpu.sync_copy(src_hbm.at[pl.ds(off, CH)], idx_s[nb])
                pltpu.sync_copy(dst_hbm.at[pl.ds(off, CH)], idx_d[nb])
                pltpu.async_copy(t_hbm.at[idx_s[nb]], rows[nb], sems[nb])

            pltpu.make_async_copy(t_hbm.at[idx_s[bb]], rows[bb], sems[bb]).wait()
            pltpu.sync_copy(rows[bb], acc.at[idx_d[bb]], add=True)

    offr = base + NCHUNK * CH
    pltpu.sync_copy(src_hbm.at[pl.ds(offr, REM)], idx_s2)
    pltpu.sync_copy(dst_hbm.at[pl.ds(offr, REM)], idx_d2)
    pltpu.async_copy(t_hbm.at[idx_s2], rows2, sem0).wait()
    pltpu.sync_copy(rows2, acc.at[idx_d2], add=True)

    plsc.subcore_barrier()

    @pl.loop(0, ZF)
    def orow(k):
        pltpu.sync_copy(acc.at[pl.ds(r0 + k * ZR, ZR)], zbuf)
        pltpu.sync_copy(zbuf, psum_hbm.at[pl.ds(cid * NPAD + r0 + k * ZR, ZR)])

    pltpu.sync_copy(acc.at[pl.ds(r0 + ZF * ZR, ZT)], zbuf.at[pl.ds(0, ZT)])
    pltpu.sync_copy(zbuf.at[pl.ds(0, ZT)],
                    psum_hbm.at[pl.ds(cid * NPAD + r0 + ZF * ZR, ZT)])


@functools.cache
def _sc_agg():
    return pl.kernel(
        _sc_agg_body,
        out_type=[jax.ShapeDtypeStruct((NC * NPAD, D), jnp.float32)],
        mesh=_mesh(),
        scratch_types=[
            pltpu.VMEM((CH,), jnp.int32),
            pltpu.VMEM((CH,), jnp.int32),
            pltpu.VMEM((CH,), jnp.int32),
            pltpu.VMEM((CH,), jnp.int32),
            pltpu.VMEM((CH, D), jnp.float32),
            pltpu.VMEM((CH, D), jnp.float32),
            pltpu.VMEM((REM,), jnp.int32),
            pltpu.VMEM((REM,), jnp.int32),
            pltpu.VMEM((REM, D), jnp.float32),
            pltpu.VMEM((ZR, D), jnp.float32),
            pltpu.VMEM_SHARED((NPAD, D), jnp.float32),
            pltpu.SemaphoreType.DMA,
            pltpu.SemaphoreType.DMA,
        ],
    )


def _mean_wb(W_ref, b_ref):
    Wbar = W_ref[0]
    bbar = b_ref[0]
    for h in range(1, H):
        Wbar = Wbar + W_ref[h]
        bbar = bbar + b_ref[h]
    return Wbar * (1.0 / H), bbar * (1.0 / H)


def _matxw(x, Wbar, bbar):
    y = lax.dot_general(x, Wbar, (((1,), (1,)), ((), ())),
                        preferred_element_type=jnp.float32)
    return y + bbar[None, :]


def _tc_transform(x, W, b):
    def body(W_ref, b_ref, x_ref, o_ref):
        Wbar, bbar = _mean_wb(W_ref, b_ref)
        o_ref[...] = _matxw(x_ref[...], Wbar, bbar)

    return pl.pallas_call(
        body,
        grid=(NBLK,),
        in_specs=[
            pl.BlockSpec((H, D, D), lambda i: (0, 0, 0)),
            pl.BlockSpec((H, D), lambda i: (0, 0)),
            pl.BlockSpec((BLK, D), lambda i: (i, 0)),
        ],
        out_specs=pl.BlockSpec((BLK, D), lambda i: (i, 0)),
        out_shape=jax.ShapeDtypeStruct((N, D), jnp.float32),
    )(W, b, x)


def _deg_inv(dp_ref):
    i = pl.program_id(0)
    deg = dp_ref[0, i] + dp_ref[1, i]
    return 1.0 / jnp.maximum(deg, 1.0)


def _tc_combine(p, dp, W, b):
    def body(W_ref, b_ref, p_ref, dp_ref, hop_ref, t_ref):
        Wbar, bbar = _mean_wb(W_ref, b_ref)
        invd = _deg_inv(dp_ref)
        hop = (p_ref[0] + p_ref[1]) * invd[:, None]
        hop_ref[...] = hop
        t_ref[...] = _matxw(hop, Wbar, bbar)

    return pl.pallas_call(
        body,
        grid=(NBLK,),
        in_specs=[
            pl.BlockSpec((H, D, D), lambda i: (0, 0, 0)),
            pl.BlockSpec((H, D), lambda i: (0, 0)),
            pl.BlockSpec((NC, BLK, D), lambda i: (0, i, 0)),
            pl.BlockSpec((NC, NBLK, BLK), lambda i: (0, 0, 0)),
        ],
        out_specs=[
            pl.BlockSpec((BLK, D), lambda i: (i, 0)),
            pl.BlockSpec((BLK, D), lambda i: (i, 0)),
        ],
        out_shape=[
            jax.ShapeDtypeStruct((N, D), jnp.float32),
            jax.ShapeDtypeStruct((N, D), jnp.float32),
        ],
    )(W, b, p, dp)


def _tc_fuse(hw, hop1, p2, dp):
    def body(hw_ref, hop1_ref, p_ref, dp_ref, o_ref):
        invd = _deg_inv(dp_ref)
        hop2 = (p_ref[0] + p_ref[1]) * invd[:, None]
        a0 = hw_ref[0]
        a1 = hw_ref[1]
        m = jnp.maximum(a0, a1)
        e0 = jnp.exp(a0 - m)
        e1 = jnp.exp(a1 - m)
        inv = 1.0 / (e0 + e1)
        o_ref[...] = (e0 * inv) * hop1_ref[...] + (e1 * inv) * hop2

    return pl.pallas_call(
        body,
        grid=(NBLK,),
        in_specs=[
            pl.BlockSpec(memory_space=pltpu.SMEM),
            pl.BlockSpec((BLK, D), lambda i: (i, 0)),
            pl.BlockSpec((NC, BLK, D), lambda i: (0, i, 0)),
            pl.BlockSpec((NC, NBLK, BLK), lambda i: (0, 0, 0)),
        ],
        out_specs=pl.BlockSpec((BLK, D), lambda i: (i, 0)),
        out_shape=jax.ShapeDtypeStruct((N, D), jnp.float32),
    )(hw, hop1, p2, dp)


@jax.jit
def kernel(node_features, edge_index, W, b, hop_weights):
    src = edge_index[0]
    dst = edge_index[1]
    dp = _sc_degree()(dst)[0].reshape(NC, NPAD)[:, :N].reshape(NC, NBLK, BLK)
    t1 = _tc_transform(node_features, W, b)
    p1 = _sc_agg()(t1, src, dst)[0].reshape(NC, NPAD, D)
    hop1, t2 = _tc_combine(p1, dp, W, b)
    p2 = _sc_agg()(t2, src, dst)[0].reshape(NC, NPAD, D)
    return _tc_fuse(hop_weights, hop1, p2, dp)


# speedup vs baseline: 37.1725x; 1.0958x over previous
import jax
import jax.numpy as jnp
from jax import lax
from jax.experimental import pallas as pl
from jax.experimental.pallas import tpu as pltpu
from jax.experimental.pallas import tpu_sc as plsc

N = 10000
E = 320000
D = 128
H = 8
NC = 2
NS = 16
NW = NC * NS
EPW = E // NW
CH = 128
NCHUNK = EPW // CH
REM = EPW - NCHUNK * CH
ZR = 64
ZF = 632 // ZR
ZT = 632 - ZF * ZR
RPS = 632
RPSB = 640
NPAD = NS * RPS
DW = 16
BLK = 1000
NBLK = N // BLK

import functools


@functools.cache
def _mesh():
    return plsc.VectorSubcoreMesh(
        core_axis_name="c", subcore_axis_name="s", num_cores=NC, num_subcores=NS
    )


def _zero_fill(ref, nrows, ncols):
    cpr = ncols // 16

    def body(i, carry):
        r = i // cpr
        c = (i % cpr) * 16
        ref[r, pl.ds(c, 16)] = jnp.zeros((16,), jnp.float32)
        return carry

    lax.fori_loop(0, nrows * cpr, body, 0)


def _one_fill(ref, nrows):
    def body(i, carry):
        ref[i, pl.ds(0, 16)] = jnp.ones((16,), jnp.float32)
        return carry

    lax.fori_loop(0, nrows, body, 0)


def _worker(cid, sid):
    base = (sid * NC + cid) * EPW
    r0 = sid * RPS
    return base, r0


def _fill_1d(ref, n, value):
    def body(i, carry):
        ref[pl.ds(i * 16, 16)] = jnp.full((16,), value, jnp.float32)
        return carry

    lax.fori_loop(0, n // 16, body, 0)


def _sc_degree_body(dst_hbm, pdeg_hbm, idx_all, ones, dbuf, dacc):
    cid = lax.axis_index("c")
    sid = lax.axis_index("s")
    base, r0 = _worker(cid, sid)

    _fill_1d(dbuf, RPSB, 0.0)
    _fill_1d(ones, CH, 1.0)
    pltpu.sync_copy(dbuf.at[pl.ds(0, RPS)], dacc.at[pl.ds(r0, RPS)])
    pltpu.sync_copy(dst_hbm.at[pl.ds(base, EPW)], idx_all)

    plsc.subcore_barrier()

    @pl.loop(0, NCHUNK)
    def chunk(j):
        pltpu.sync_copy(ones, dacc.at[idx_all.at[pl.ds(j * CH, CH)]], add=True)

    pltpu.sync_copy(ones.at[pl.ds(0, REM)],
                    dacc.at[idx_all.at[pl.ds(NCHUNK * CH, REM)]], add=True)

    plsc.subcore_barrier()

    pltpu.sync_copy(dacc.at[pl.ds(r0, RPS)], dbuf.at[pl.ds(0, RPS)])
    pltpu.sync_copy(dbuf.at[pl.ds(0, RPS)],
                    pdeg_hbm.at[pl.ds(cid * NPAD + r0, RPS)])


@functools.cache
def _sc_degree():
    return pl.kernel(
        _sc_degree_body,
        out_type=[jax.ShapeDtypeStruct((NC * NPAD,), jnp.float32)],
        mesh=_mesh(),
        scratch_types=[
            pltpu.VMEM((EPW,), jnp.int32),
            pltpu.VMEM((CH,), jnp.float32),
            pltpu.VMEM((RPSB,), jnp.float32),
            pltpu.VMEM_SHARED((NPAD,), jnp.float32),
        ],
    )


def _sc_agg_body(t_hbm, src_hbm, dst_hbm, psum_hbm,
                 idx_s0, idx_d0, idx_s1, idx_d1, rows0, rows1,
                 idx_s2, idx_d2, rows2, zbuf, acc, sem0, sem1):
    cid = lax.axis_index("c")
    sid = lax.axis_index("s")
    base, r0 = _worker(cid, sid)
    idx_s = (idx_s0, idx_s1)
    idx_d = (idx_d0, idx_d1)
    rows = (rows0, rows1)
    sems = (sem0, sem1)

    _zero_fill(zbuf, ZR, D)

    @pl.loop(0, ZF)
    def zrow(k):
        pltpu.sync_copy(zbuf, acc.at[pl.ds(r0 + k * ZR, ZR)])

    pltpu.sync_copy(zbuf.at[pl.ds(0, ZT)], acc.at[pl.ds(r0 + ZF * ZR, ZT)])

    plsc.subcore_barrier()

    pltpu.sync_copy(src_hbm.at[pl.ds(base, CH)], idx_s0)
    pltpu.sync_copy(dst_hbm.at[pl.ds(base, CH)], idx_d0)
    pltpu.async_copy(t_hbm.at[idx_s0], rows0, sem0)

    @pl.loop(0, NCHUNK, step=2)
    def pair(i):
        for bb in range(2):
            j = i + bb
            nb = 1 - bb
            jn = j + 1

            @pl.when(jn < NCHUNK)
            def _():
                off = base + jn * CH
                pltpu.sync_copy(src_hbm.at[pl.ds(off, CH)], idx_s[nb])
                pltpu.sync_copy(dst_hbm.at[pl.ds(off, CH)], idx_d[nb])
                pltpu.async_copy(t_hbm.at[idx_s[nb]], rows[nb], sems[nb])

            pltpu.make_async_copy(t_hbm.at[idx_s[bb]], rows[bb], sems[bb]).wait()
            pltpu.sync_copy(rows[bb], acc.at[idx_d[bb]], add=True)

    offr = base + NCHUNK * CH
    pltpu.sync_copy(src_hbm.at[pl.ds(offr, REM)], idx_s2)
    pltpu.sync_copy(dst_hbm.at[pl.ds(offr, REM)], idx_d2)
    pltpu.async_copy(t_hbm.at[idx_s2], rows2, sem0).wait()
    pltpu.sync_copy(rows2, acc.at[idx_d2], add=True)

    plsc.subcore_barrier()

    @pl.loop(0, ZF)
    def orow(k):
        pltpu.sync_copy(acc.at[pl.ds(r0 + k * ZR, ZR)], zbuf)
        pltpu.sync_copy(zbuf, psum_hbm.at[pl.ds(cid * NPAD + r0 + k * ZR, ZR)])

    pltpu.sync_copy(acc.at[pl.ds(r0 + ZF * ZR, ZT)], zbuf.at[pl.ds(0, ZT)])
    pltpu.sync_copy(zbuf.at[pl.ds(0, ZT)],
                    psum_hbm.at[pl.ds(cid * NPAD + r0 + ZF * ZR, ZT)])


@functools.cache
def _sc_agg():
    return pl.kernel(
        _sc_agg_body,
        out_type=[jax.ShapeDtypeStruct((NC * NPAD, D), jnp.float32)],
        mesh=_mesh(),
        scratch_types=[
            pltpu.VMEM((CH,), jnp.int32),
            pltpu.VMEM((CH,), jnp.int32),
            pltpu.VMEM((CH,), jnp.int32),
            pltpu.VMEM((CH,), jnp.int32),
            pltpu.VMEM((CH, D), jnp.float32),
            pltpu.VMEM((CH, D), jnp.float32),
            pltpu.VMEM((REM,), jnp.int32),
            pltpu.VMEM((REM,), jnp.int32),
            pltpu.VMEM((REM, D), jnp.float32),
            pltpu.VMEM((ZR, D), jnp.float32),
            pltpu.VMEM_SHARED((NPAD, D), jnp.float32),
            pltpu.SemaphoreType.DMA,
            pltpu.SemaphoreType.DMA,
        ],
    )


def _mean_wb(W_ref, b_ref):
    Wbar = W_ref[0]
    bbar = b_ref[0]
    for h in range(1, H):
        Wbar = Wbar + W_ref[h]
        bbar = bbar + b_ref[h]
    return Wbar * (1.0 / H), bbar * (1.0 / H)


def _matxw(x, Wbar, bbar):
    y = lax.dot_general(x, Wbar, (((1,), (1,)), ((), ())),
                        preferred_element_type=jnp.float32)
    return y + bbar[None, :]


def _tc_transform(x, W, b):
    def body(W_ref, b_ref, x_ref, o_ref):
        Wbar, bbar = _mean_wb(W_ref, b_ref)
        o_ref[...] = _matxw(x_ref[...], Wbar, bbar)

    return pl.pallas_call(
        body,
        grid=(NBLK,),
        in_specs=[
            pl.BlockSpec((H, D, D), lambda i: (0, 0, 0)),
            pl.BlockSpec((H, D), lambda i: (0, 0)),
            pl.BlockSpec((BLK, D), lambda i: (i, 0)),
        ],
        out_specs=pl.BlockSpec((BLK, D), lambda i: (i, 0)),
        out_shape=jax.ShapeDtypeStruct((N, D), jnp.float32),
    )(W, b, x)


def _deg_inv(dp_ref):
    i = pl.program_id(0)
    deg = dp_ref[0, i] + dp_ref[1, i]
    return 1.0 / jnp.maximum(deg, 1.0)


def _tc_combine(p, dp, W, b):
    def body(W_ref, b_ref, p_ref, dp_ref, hop_ref, t_ref):
        Wbar, bbar = _mean_wb(W_ref, b_ref)
        invd = _deg_inv(dp_ref)
        hop = (p_ref[0] + p_ref[1]) * invd[:, None]
        hop_ref[...] = hop
        t_ref[...] = _matxw(hop, Wbar, bbar)

    return pl.pallas_call(
        body,
        grid=(NBLK,),
        in_specs=[
            pl.BlockSpec((H, D, D), lambda i: (0, 0, 0)),
            pl.BlockSpec((H, D), lambda i: (0, 0)),
            pl.BlockSpec((NC, BLK, D), lambda i: (0, i, 0)),
            pl.BlockSpec((NC, NBLK, BLK), lambda i: (0, 0, 0)),
        ],
        out_specs=[
            pl.BlockSpec((BLK, D), lambda i: (i, 0)),
            pl.BlockSpec((BLK, D), lambda i: (i, 0)),
        ],
        out_shape=[
            jax.ShapeDtypeStruct((N, D), jnp.float32),
            jax.ShapeDtypeStruct((N, D), jnp.float32),
        ],
    )(W, b, p, dp)


def _tc_fuse(hw, hop1, p2, dp):
    def body(hw_ref, hop1_ref, p_ref, dp_ref, o_ref):
        invd = _deg_inv(dp_ref)
        hop2 = (p_ref[0] + p_ref[1]) * invd[:, None]
        a0 = hw_ref[0]
        a1 = hw_ref[1]
        m = jnp.maximum(a0, a1)
        e0 = jnp.exp(a0 - m)
        e1 = jnp.exp(a1 - m)
        inv = 1.0 / (e0 + e1)
        o_ref[...] = (e0 * inv) * hop1_ref[...] + (e1 * inv) * hop2

    return pl.pallas_call(
        body,
        grid=(NBLK,),
        in_specs=[
            pl.BlockSpec(memory_space=pltpu.SMEM),
            pl.BlockSpec((BLK, D), lambda i: (i, 0)),
            pl.BlockSpec((NC, BLK, D), lambda i: (0, i, 0)),
            pl.BlockSpec((NC, NBLK, BLK), lambda i: (0, 0, 0)),
        ],
        out_specs=pl.BlockSpec((BLK, D), lambda i: (i, 0)),
        out_shape=jax.ShapeDtypeStruct((N, D), jnp.float32),
    )(hw, hop1, p2, dp)


@jax.jit
def kernel(node_features, edge_index, W, b, hop_weights):
    src = edge_index[0]
    dst = edge_index[1]
    dp = _sc_degree()(dst)[0].reshape(NC, NPAD)[:, :N].reshape(NC, NBLK, BLK)
    t1 = _tc_transform(node_features, W, b)
    p1 = _sc_agg()(t1, src, dst)[0].reshape(NC, NPAD, D)
    hop1, t2 = _tc_combine(p1, dp, W, b)
    p2 = _sc_agg()(t2, src, dst)[0].reshape(NC, NPAD, D)
    return _tc_fuse(hop_weights, hop1, p2, dp)


# speedup vs baseline: 44.3577x; 1.1933x over previous
import jax
import jax.numpy as jnp
from jax import lax
from jax.experimental import pallas as pl
from jax.experimental.pallas import tpu as pltpu
from jax.experimental.pallas import tpu_sc as plsc

N = 10000
E = 320000
D = 128
H = 8
NC = 2
NS = 16
NW = NC * NS
EPW = E // NW
CH = 128
NCHUNK = EPW // CH
REM = EPW - NCHUNK * CH
SB = 2048
SBB = SB // CH
NSB = EPW // SB
TAIL = EPW - NSB * SB
TBATCH = TAIL // CH
ZR = 64
ZF = 632 // ZR
ZT = 632 - ZF * ZR
RPS = 632
RPSB = 640
NPAD = NS * RPS
DW = 16
BLK = 1000
NBLK = N // BLK

import functools


@functools.cache
def _mesh():
    return plsc.VectorSubcoreMesh(
        core_axis_name="c", subcore_axis_name="s", num_cores=NC, num_subcores=NS
    )


def _zero_fill(ref, nrows, ncols):
    cpr = ncols // 16

    def body(i, carry):
        r = i // cpr
        c = (i % cpr) * 16
        ref[r, pl.ds(c, 16)] = jnp.zeros((16,), jnp.float32)
        return carry

    lax.fori_loop(0, nrows * cpr, body, 0)


def _one_fill(ref, nrows):
    def body(i, carry):
        ref[i, pl.ds(0, 16)] = jnp.ones((16,), jnp.float32)
        return carry

    lax.fori_loop(0, nrows, body, 0)


def _worker(cid, sid):
    base = (sid * NC + cid) * EPW
    r0 = sid * RPS
    return base, r0


def _fill_1d(ref, n, value):
    def body(i, carry):
        ref[pl.ds(i * 16, 16)] = jnp.full((16,), value, jnp.float32)
        return carry

    lax.fori_loop(0, n // 16, body, 0)


def _sc_degree_body(dst_hbm, pdeg_hbm, idx_all, ones, dbuf, dacc):
    cid = lax.axis_index("c")
    sid = lax.axis_index("s")
    base, r0 = _worker(cid, sid)

    _fill_1d(dbuf, RPSB, 0.0)
    _fill_1d(ones, CH, 1.0)
    pltpu.sync_copy(dbuf.at[pl.ds(0, RPS)], dacc.at[pl.ds(r0, RPS)])
    pltpu.sync_copy(dst_hbm.at[pl.ds(base, EPW)], idx_all)

    plsc.subcore_barrier()

    @pl.loop(0, NCHUNK)
    def chunk(j):
        pltpu.sync_copy(ones, dacc.at[idx_all.at[pl.ds(j * CH, CH)]], add=True)

    pltpu.sync_copy(ones.at[pl.ds(0, REM)],
                    dacc.at[idx_all.at[pl.ds(NCHUNK * CH, REM)]], add=True)

    plsc.subcore_barrier()

    pltpu.sync_copy(dacc.at[pl.ds(r0, RPS)], dbuf.at[pl.ds(0, RPS)])
    pltpu.sync_copy(dbuf.at[pl.ds(0, RPS)],
                    pdeg_hbm.at[pl.ds(cid * NPAD + r0, RPS)])


@functools.cache
def _sc_degree():
    return pl.kernel(
        _sc_degree_body,
        out_type=[jax.ShapeDtypeStruct((NC * NPAD,), jnp.float32)],
        mesh=_mesh(),
        scratch_types=[
            pltpu.VMEM((EPW,), jnp.int32),
            pltpu.VMEM((CH,), jnp.float32),
            pltpu.VMEM((RPSB,), jnp.float32),
            pltpu.VMEM_SHARED((NPAD,), jnp.float32),
        ],
    )


def _sc_agg_body(t_hbm, src_hbm, dst_hbm, psum_hbm,
                 sbuf_s, sbuf_d, rows0, rows1, rows2, zbuf, acc, sem0, sem1):
    cid = lax.axis_index("c")
    sid = lax.axis_index("s")
    base, r0 = _worker(cid, sid)
    rows = (rows0, rows1)
    sems = (sem0, sem1)

    _zero_fill(zbuf, ZR, D)

    @pl.loop(0, ZF)
    def zrow(k):
        pltpu.sync_copy(zbuf, acc.at[pl.ds(r0 + k * ZR, ZR)])

    pltpu.sync_copy(zbuf.at[pl.ds(0, ZT)], acc.at[pl.ds(r0 + ZF * ZR, ZT)])

    plsc.subcore_barrier()

    def run_block(nbatch):
        pltpu.async_copy(t_hbm.at[sbuf_s.at[pl.ds(0, CH)]], rows0, sem0)

        @pl.loop(0, nbatch, step=2)
        def pair(i):
            for bb in range(2):
                j = i + bb
                nb = 1 - bb
                jn = j + 1

                @pl.when(jn < nbatch)
                def _():
                    pltpu.async_copy(
                        t_hbm.at[sbuf_s.at[pl.ds(jn * CH, CH)]],
                        rows[nb], sems[nb])

                pltpu.make_async_copy(
                    t_hbm.at[sbuf_s.at[pl.ds(j * CH, CH)]],
                    rows[bb], sems[bb]).wait()
                pltpu.sync_copy(rows[bb],
                                acc.at[sbuf_d.at[pl.ds(j * CH, CH)]],
                                add=True)

    @pl.loop(0, NSB)
    def sbatch(ss):
        off = base + ss * SB
        pltpu.sync_copy(src_hbm.at[pl.ds(off, SB)], sbuf_s)
        pltpu.sync_copy(dst_hbm.at[pl.ds(off, SB)], sbuf_d)
        run_block(SBB)

    offt = base + NSB * SB
    pltpu.sync_copy(src_hbm.at[pl.ds(offt, TAIL)], sbuf_s.at[pl.ds(0, TAIL)])
    pltpu.sync_copy(dst_hbm.at[pl.ds(offt, TAIL)], sbuf_d.at[pl.ds(0, TAIL)])
    run_block(TBATCH)

    pltpu.async_copy(t_hbm.at[sbuf_s.at[pl.ds(TBATCH * CH, REM)]],
                     rows2, sem0).wait()
    pltpu.sync_copy(rows2, acc.at[sbuf_d.at[pl.ds(TBATCH * CH, REM)]],
                    add=True)

    plsc.subcore_barrier()

    @pl.loop(0, ZF)
    def orow(k):
        pltpu.sync_copy(acc.at[pl.ds(r0 + k * ZR, ZR)], zbuf)
        pltpu.sync_copy(zbuf, psum_hbm.at[pl.ds(cid * NPAD + r0 + k * ZR, ZR)])

    pltpu.sync_copy(acc.at[pl.ds(r0 + ZF * ZR, ZT)], zbuf.at[pl.ds(0, ZT)])
    pltpu.sync_copy(zbuf.at[pl.ds(0, ZT)],
                    psum_hbm.at[pl.ds(cid * NPAD + r0 + ZF * ZR, ZT)])


@functools.cache
def _sc_agg():
    return pl.kernel(
        _sc_agg_body,
        out_type=[jax.ShapeDtypeStruct((NC * NPAD, D), jnp.float32)],
        mesh=_mesh(),
        scratch_types=[
            pltpu.VMEM((SB,), jnp.int32),
            pltpu.VMEM((SB,), jnp.int32),
            pltpu.VMEM((CH, D), jnp.float32),
            pltpu.VMEM((CH, D), jnp.float32),
            pltpu.VMEM((REM, D), jnp.float32),
            pltpu.VMEM((ZR, D), jnp.float32),
            pltpu.VMEM_SHARED((NPAD, D), jnp.float32),
            pltpu.SemaphoreType.DMA,
            pltpu.SemaphoreType.DMA,
        ],
    )


def _mean_wb(W_ref, b_ref):
    Wbar = W_ref[0]
    bbar = b_ref[0]
    for h in range(1, H):
        Wbar = Wbar + W_ref[h]
        bbar = bbar + b_ref[h]
    return Wbar * (1.0 / H), bbar * (1.0 / H)


def _matxw(x, Wbar, bbar):
    y = lax.dot_general(x, Wbar, (((1,), (1,)), ((), ())),
                        preferred_element_type=jnp.float32)
    return y + bbar[None, :]


def _tc_transform(x, W, b):
    def body(W_ref, b_ref, x_ref, o_ref):
        Wbar, bbar = _mean_wb(W_ref, b_ref)
        o_ref[...] = _matxw(x_ref[...], Wbar, bbar)

    return pl.pallas_call(
        body,
        grid=(NBLK,),
        in_specs=[
            pl.BlockSpec((H, D, D), lambda i: (0, 0, 0)),
            pl.BlockSpec((H, D), lambda i: (0, 0)),
            pl.BlockSpec((BLK, D), lambda i: (i, 0)),
        ],
        out_specs=pl.BlockSpec((BLK, D), lambda i: (i, 0)),
        out_shape=jax.ShapeDtypeStruct((N, D), jnp.float32),
    )(W, b, x)


def _deg_inv(dp_ref):
    i = pl.program_id(0)
    deg = dp_ref[0, i] + dp_ref[1, i]
    return 1.0 / jnp.maximum(deg, 1.0)


def _tc_combine(p, dp, W, b):
    def body(W_ref, b_ref, p_ref, dp_ref, hop_ref, t_ref):
        Wbar, bbar = _mean_wb(W_ref, b_ref)
        invd = _deg_inv(dp_ref)
        hop = (p_ref[0] + p_ref[1]) * invd[:, None]
        hop_ref[...] = hop
        t_ref[...] = _matxw(hop, Wbar, bbar)

    return pl.pallas_call(
        body,
        grid=(NBLK,),
        in_specs=[
            pl.BlockSpec((H, D, D), lambda i: (0, 0, 0)),
            pl.BlockSpec((H, D), lambda i: (0, 0)),
            pl.BlockSpec((NC, BLK, D), lambda i: (0, i, 0)),
            pl.BlockSpec((NC, NBLK, BLK), lambda i: (0, 0, 0)),
        ],
        out_specs=[
            pl.BlockSpec((BLK, D), lambda i: (i, 0)),
            pl.BlockSpec((BLK, D), lambda i: (i, 0)),
        ],
        out_shape=[
            jax.ShapeDtypeStruct((N, D), jnp.float32),
            jax.ShapeDtypeStruct((N, D), jnp.float32),
        ],
    )(W, b, p, dp)


def _tc_fuse(hw, hop1, p2, dp):
    def body(hw_ref, hop1_ref, p_ref, dp_ref, o_ref):
        invd = _deg_inv(dp_ref)
        hop2 = (p_ref[0] + p_ref[1]) * invd[:, None]
        a0 = hw_ref[0]
        a1 = hw_ref[1]
        m = jnp.maximum(a0, a1)
        e0 = jnp.exp(a0 - m)
        e1 = jnp.exp(a1 - m)
        inv = 1.0 / (e0 + e1)
        o_ref[...] = (e0 * inv) * hop1_ref[...] + (e1 * inv) * hop2

    return pl.pallas_call(
        body,
        grid=(NBLK,),
        in_specs=[
            pl.BlockSpec(memory_space=pltpu.SMEM),
            pl.BlockSpec((BLK, D), lambda i: (i, 0)),
            pl.BlockSpec((NC, BLK, D), lambda i: (0, i, 0)),
            pl.BlockSpec((NC, NBLK, BLK), lambda i: (0, 0, 0)),
        ],
        out_specs=pl.BlockSpec((BLK, D), lambda i: (i, 0)),
        out_shape=jax.ShapeDtypeStruct((N, D), jnp.float32),
    )(hw, hop1, p2, dp)


@jax.jit
def kernel(node_features, edge_index, W, b, hop_weights):
    src = edge_index[0]
    dst = edge_index[1]
    dp = _sc_degree()(dst)[0].reshape(NC, NPAD)[:, :N].reshape(NC, NBLK, BLK)
    t1 = _tc_transform(node_features, W, b)
    p1 = _sc_agg()(t1, src, dst)[0].reshape(NC, NPAD, D)
    hop1, t2 = _tc_combine(p1, dp, W, b)
    p2 = _sc_agg()(t2, src, dst)[0].reshape(NC, NPAD, D)
    return _tc_fuse(hop_weights, hop1, p2, dp)


# speedup vs baseline: 44.9215x; 1.0127x over previous
import jax
import jax.numpy as jnp
from jax import lax
from jax.experimental import pallas as pl
from jax.experimental.pallas import tpu as pltpu
from jax.experimental.pallas import tpu_sc as plsc

N = 10000
E = 320000
D = 128
H = 8
NC = 2
NS = 16
NW = NC * NS
EPW = E // NW
CH = 128
NCHUNK = EPW // CH
REM = EPW - NCHUNK * CH
SB = 2048
SBB = SB // CH
NSB = EPW // SB
TAIL = EPW - NSB * SB
TBATCH = TAIL // CH
ZR = 64
ZF = 632 // ZR
ZT = 632 - ZF * ZR
OZ = 32
OF = 632 // OZ
OT = 632 - OF * OZ
RPS = 632
RPSB = 640
NPAD = NS * RPS
DW = 16
BLK = 1000
NBLK = N // BLK

import functools


@functools.cache
def _mesh():
    return plsc.VectorSubcoreMesh(
        core_axis_name="c", subcore_axis_name="s", num_cores=NC, num_subcores=NS
    )


def _zero_fill(ref, nrows, ncols):
    cpr = ncols // 16

    def body(i, carry):
        r = i // cpr
        c = (i % cpr) * 16
        ref[r, pl.ds(c, 16)] = jnp.zeros((16,), jnp.float32)
        return carry

    lax.fori_loop(0, nrows * cpr, body, 0)


def _one_fill(ref, nrows):
    def body(i, carry):
        ref[i, pl.ds(0, 16)] = jnp.ones((16,), jnp.float32)
        return carry

    lax.fori_loop(0, nrows, body, 0)


def _worker(cid, sid):
    base = (sid * NC + cid) * EPW
    r0 = sid * RPS
    return base, r0


def _fill_1d(ref, n, value):
    def body(i, carry):
        ref[pl.ds(i * 16, 16)] = jnp.full((16,), value, jnp.float32)
        return carry

    lax.fori_loop(0, n // 16, body, 0)


def _sc_degree_body(dst_hbm, pdeg_hbm, idx_all, ones, dbuf, dacc):
    cid = lax.axis_index("c")
    sid = lax.axis_index("s")
    base, r0 = _worker(cid, sid)

    _fill_1d(dbuf, RPSB, 0.0)
    _fill_1d(ones, CH, 1.0)
    pltpu.sync_copy(dbuf.at[pl.ds(0, RPS)], dacc.at[pl.ds(r0, RPS)])
    pltpu.sync_copy(dst_hbm.at[pl.ds(base, EPW)], idx_all)

    plsc.subcore_barrier()

    @pl.loop(0, NCHUNK)
    def chunk(j):
        pltpu.sync_copy(ones, dacc.at[idx_all.at[pl.ds(j * CH, CH)]], add=True)

    pltpu.sync_copy(ones.at[pl.ds(0, REM)],
                    dacc.at[idx_all.at[pl.ds(NCHUNK * CH, REM)]], add=True)

    plsc.subcore_barrier()

    pltpu.sync_copy(dacc.at[pl.ds(r0, RPS)], dbuf.at[pl.ds(0, RPS)])
    pltpu.sync_copy(dbuf.at[pl.ds(0, RPS)],
                    pdeg_hbm.at[pl.ds(cid * NPAD + r0, RPS)])


@functools.cache
def _sc_degree():
    return pl.kernel(
        _sc_degree_body,
        out_type=[jax.ShapeDtypeStruct((NC * NPAD,), jnp.float32)],
        mesh=_mesh(),
        scratch_types=[
            pltpu.VMEM((EPW,), jnp.int32),
            pltpu.VMEM((CH,), jnp.float32),
            pltpu.VMEM((RPSB,), jnp.float32),
            pltpu.VMEM_SHARED((NPAD,), jnp.float32),
        ],
    )


def _sc_agg_body(t_hbm, src_hbm, dst_hbm, psum_hbm,
                 sbuf_s, sbuf_d, rows0, rows1, rows2, zbuf, acc, sem0, sem1):
    cid = lax.axis_index("c")
    sid = lax.axis_index("s")
    base, r0 = _worker(cid, sid)
    rows = (rows0, rows1)
    sems = (sem0, sem1)

    _zero_fill(zbuf, ZR, D)

    @pl.loop(0, ZF)
    def zrow(k):
        pltpu.sync_copy(zbuf, acc.at[pl.ds(r0 + k * ZR, ZR)])

    pltpu.sync_copy(zbuf.at[pl.ds(0, ZT)], acc.at[pl.ds(r0 + ZF * ZR, ZT)])

    plsc.subcore_barrier()

    def run_block(nbatch):
        pltpu.async_copy(t_hbm.at[sbuf_s.at[pl.ds(0, CH)]], rows0, sem0)

        @pl.loop(0, nbatch, step=2)
        def pair(i):
            for bb in range(2):
                j = i + bb
                nb = 1 - bb
                jn = j + 1

                @pl.when(jn < nbatch)
                def _():
                    pltpu.async_copy(
                        t_hbm.at[sbuf_s.at[pl.ds(jn * CH, CH)]],
                        rows[nb], sems[nb])

                pltpu.make_async_copy(
                    t_hbm.at[sbuf_s.at[pl.ds(j * CH, CH)]],
                    rows[bb], sems[bb]).wait()
                pltpu.sync_copy(rows[bb],
                                acc.at[sbuf_d.at[pl.ds(j * CH, CH)]],
                                add=True)

    @pl.loop(0, NSB)
    def sbatch(ss):
        off = base + ss * SB
        pltpu.sync_copy(src_hbm.at[pl.ds(off, SB)], sbuf_s)
        pltpu.sync_copy(dst_hbm.at[pl.ds(off, SB)], sbuf_d)
        run_block(SBB)

    offt = base + NSB * SB
    pltpu.sync_copy(src_hbm.at[pl.ds(offt, TAIL)], sbuf_s.at[pl.ds(0, TAIL)])
    pltpu.sync_copy(dst_hbm.at[pl.ds(offt, TAIL)], sbuf_d.at[pl.ds(0, TAIL)])
    run_block(TBATCH)

    pltpu.async_copy(t_hbm.at[sbuf_s.at[pl.ds(TBATCH * CH, REM)]],
                     rows2, sem0).wait()
    pltpu.sync_copy(rows2, acc.at[sbuf_d.at[pl.ds(TBATCH * CH, REM)]],
                    add=True)

    plsc.subcore_barrier()

    def out_unit(k, h, nrows):
        pltpu.sync_copy(acc.at[pl.ds(r0 + k * OZ, nrows)],
                        zbuf.at[pl.ds(h * OZ, nrows)])
        pltpu.async_copy(zbuf.at[pl.ds(h * OZ, nrows)],
                         psum_hbm.at[pl.ds(cid * NPAD + r0 + k * OZ, nrows)],
                         sems[h])

    def out_wait(k, h, nrows):
        pltpu.make_async_copy(
            zbuf.at[pl.ds(h * OZ, nrows)],
            psum_hbm.at[pl.ds(cid * NPAD + r0 + k * OZ, nrows)],
            sems[h]).wait()

    @pl.loop(0, OF - 1, step=2)
    def ounit(i):
        for bb in range(2):
            k = i + bb

            @pl.when(k >= 2)
            def _():
                out_wait(k - 2, bb, OZ)

            out_unit(k, bb, OZ)

    out_wait(OF - 3, 0, OZ)
    out_unit(OF - 1, 0, OZ)
    out_wait(OF - 2, 1, OZ)
    pltpu.sync_copy(acc.at[pl.ds(r0 + OF * OZ, OT)], zbuf.at[pl.ds(OZ, OT)])
    pltpu.async_copy(zbuf.at[pl.ds(OZ, OT)],
                     psum_hbm.at[pl.ds(cid * NPAD + r0 + OF * OZ, OT)],
                     sem1)
    out_wait(OF - 1, 0, OZ)
    pltpu.make_async_copy(
        zbuf.at[pl.ds(OZ, OT)],
        psum_hbm.at[pl.ds(cid * NPAD + r0 + OF * OZ, OT)],
        sem1).wait()


@functools.cache
def _sc_agg():
    return pl.kernel(
        _sc_agg_body,
        out_type=[jax.ShapeDtypeStruct((NC * NPAD, D), jnp.float32)],
        mesh=_mesh(),
        scratch_types=[
            pltpu.VMEM((SB,), jnp.int32),
            pltpu.VMEM((SB,), jnp.int32),
            pltpu.VMEM((CH, D), jnp.float32),
            pltpu.VMEM((CH, D), jnp.float32),
            pltpu.VMEM((REM, D), jnp.float32),
            pltpu.VMEM((ZR, D), jnp.float32),
            pltpu.VMEM_SHARED((NPAD, D), jnp.float32),
            pltpu.SemaphoreType.DMA,
            pltpu.SemaphoreType.DMA,
        ],
    )


def _mean_wb(W_ref, b_ref):
    Wbar = W_ref[0]
    bbar = b_ref[0]
    for h in range(1, H):
        Wbar = Wbar + W_ref[h]
        bbar = bbar + b_ref[h]
    return Wbar * (1.0 / H), bbar * (1.0 / H)


def _matxw(x, Wbar, bbar):
    y = lax.dot_general(x, Wbar, (((1,), (1,)), ((), ())),
                        preferred_element_type=jnp.float32)
    return y + bbar[None, :]


def _tc_transform(x, W, b):
    def body(W_ref, b_ref, x_ref, o_ref):
        Wbar, bbar = _mean_wb(W_ref, b_ref)
        o_ref[...] = _matxw(x_ref[...], Wbar, bbar)

    return pl.pallas_call(
        body,
        grid=(NBLK,),
        in_specs=[
            pl.BlockSpec((H, D, D), lambda i: (0, 0, 0)),
            pl.BlockSpec((H, D), lambda i: (0, 0)),
            pl.BlockSpec((BLK, D), lambda i: (i, 0)),
        ],
        out_specs=pl.BlockSpec((BLK, D), lambda i: (i, 0)),
        out_shape=jax.ShapeDtypeStruct((N, D), jnp.float32),
    )(W, b, x)


def _deg_inv(dp_ref):
    i = pl.program_id(0)
    deg = dp_ref[0, i] + dp_ref[1, i]
    return 1.0 / jnp.maximum(deg, 1.0)


def _tc_combine(p, dp, W, b):
    def body(W_ref, b_ref, p_ref, dp_ref, hop_ref, t_ref):
        Wbar, bbar = _mean_wb(W_ref, b_ref)
        invd = _deg_inv(dp_ref)
        hop = (p_ref[0] + p_ref[1]) * invd[:, None]
        hop_ref[...] = hop
        t_ref[...] = _matxw(hop, Wbar, bbar)

    return pl.pallas_call(
        body,
        grid=(NBLK,),
        in_specs=[
            pl.BlockSpec((H, D, D), lambda i: (0, 0, 0)),
            pl.BlockSpec((H, D), lambda i: (0, 0)),
            pl.BlockSpec((NC, BLK, D), lambda i: (0, i, 0)),
            pl.BlockSpec((NC, NBLK, BLK), lambda i: (0, 0, 0)),
        ],
        out_specs=[
            pl.BlockSpec((BLK, D), lambda i: (i, 0)),
            pl.BlockSpec((BLK, D), lambda i: (i, 0)),
        ],
        out_shape=[
            jax.ShapeDtypeStruct((N, D), jnp.float32),
            jax.ShapeDtypeStruct((N, D), jnp.float32),
        ],
    )(W, b, p, dp)


def _tc_fuse(hw, hop1, p2, dp):
    def body(hw_ref, hop1_ref, p_ref, dp_ref, o_ref):
        invd = _deg_inv(dp_ref)
        hop2 = (p_ref[0] + p_ref[1]) * invd[:, None]
        a0 = hw_ref[0]
        a1 = hw_ref[1]
        m = jnp.maximum(a0, a1)
        e0 = jnp.exp(a0 - m)
        e1 = jnp.exp(a1 - m)
        inv = 1.0 / (e0 + e1)
        o_ref[...] = (e0 * inv) * hop1_ref[...] + (e1 * inv) * hop2

    return pl.pallas_call(
        body,
        grid=(NBLK,),
        in_specs=[
            pl.BlockSpec(memory_space=pltpu.SMEM),
            pl.BlockSpec((BLK, D), lambda i: (i, 0)),
            pl.BlockSpec((NC, BLK, D), lambda i: (0, i, 0)),
            pl.BlockSpec((NC, NBLK, BLK), lambda i: (0, 0, 0)),
        ],
        out_specs=pl.BlockSpec((BLK, D), lambda i: (i, 0)),
        out_shape=jax.ShapeDtypeStruct((N, D), jnp.float32),
    )(hw, hop1, p2, dp)


@jax.jit
def kernel(node_features, edge_index, W, b, hop_weights):
    src = edge_index[0]
    dst = edge_index[1]
    dp = _sc_degree()(dst)[0].reshape(NC, NPAD)[:, :N].reshape(NC, NBLK, BLK)
    t1 = _tc_transform(node_features, W, b)
    p1 = _sc_agg()(t1, src, dst)[0].reshape(NC, NPAD, D)
    hop1, t2 = _tc_combine(p1, dp, W, b)
    p2 = _sc_agg()(t2, src, dst)[0].reshape(NC, NPAD, D)
    return _tc_fuse(hop_weights, hop1, p2, dp)
